# baseline (speedup 1.0000x reference)
import functools

import jax
import jax.numpy as jnp
from jax import lax
from jax.experimental import pallas as pl
from jax.experimental.pallas import tpu as pltpu
from jax.experimental.pallas import tpu_sc as plsc

BODIES = 16
NVERT = 4096
NFACE = 8192
KNBR = 8
SIGMA = 0.5
COLL_W = 1.0

NC, NS, LANES = 2, 16, 16
NW = NC * NS
FPW = NFACE // NW
NGRP = FPW // LANES
QBLK = 256
BIG = 1e30


def _sc_mesh():
    return plsc.VectorSubcoreMesh(
        core_axis_name="c", subcore_axis_name="s",
        num_cores=NC, num_subcores=NS)


def _prep_body(verts_hbm, faces_hbm, qmat_hbm, centT_hbm, tris_hbm,
               vbuf, fbuf, qbuf, ctbuf, tbuf):
    w = lax.axis_index("s") * NC + lax.axis_index("c")
    f0 = w * FPW
    lane = lax.iota(jnp.int32, LANES)
    one = jnp.full((LANES,), 1.0, jnp.float32)

    pltpu.sync_copy(faces_hbm.at[pl.ds(f0 * 3, FPW * 3)], fbuf)

    def per_body(b, carry):
        pltpu.sync_copy(verts_hbm.at[b], vbuf)

        def per_group(gi, carry2):
            fl = gi * LANES + lane
            vcoord = []
            for j in range(3):
                ij = plsc.load_gather(fbuf, [fl * 3 + j])
                vcoord.append([
                    plsc.load_gather(vbuf, [ij, jnp.full((LANES,), c, jnp.int32)])
                    for c in range(3)])
            cent = [(vcoord[0][c] + vcoord[1][c] + vcoord[2][c]) * (1.0 / 3.0)
                    for c in range(3)]
            nc2 = 0.5 * (cent[0] * cent[0] + cent[1] * cent[1]
                         + cent[2] * cent[2])
            for col, v in enumerate([-cent[0], -cent[1], -cent[2], one]):
                plsc.store_scatter(
                    qbuf, [fl, jnp.full((LANES,), col, jnp.int32)], v)
            for r, v in enumerate([cent[0], cent[1], cent[2], nc2]):
                plsc.store_scatter(
                    ctbuf, [jnp.full((LANES,), r, jnp.int32), fl], v)
            for j in range(3):
                for c in range(3):
                    plsc.store_scatter(
                        tbuf, [fl, jnp.full((LANES,), 3 * j + c, jnp.int32)],
                        vcoord[j][c])
            for c in range(3):
                plsc.store_scatter(
                    tbuf, [fl, jnp.full((LANES,), 12 + c, jnp.int32)], cent[c])
            return carry2

        lax.fori_loop(0, NGRP, per_group, 0)
        pltpu.sync_copy(qbuf, qmat_hbm.at[b, pl.ds(f0, FPW)])
        pltpu.sync_copy(ctbuf, centT_hbm.at[b, :, pl.ds(f0, FPW)])
        pltpu.sync_copy(tbuf, tris_hbm.at[pl.ds(b * NFACE + f0, FPW)])
        return carry

    lax.fori_loop(0, BODIES, per_body, 0)


@functools.cache
def _prep():
    return pl.kernel(
        _prep_body,
        out_type=[pltpu.HBM((BODIES, NFACE, 4), jnp.float32),
                  pltpu.HBM((BODIES, 4, NFACE), jnp.float32),
                  pltpu.HBM((BODIES * NFACE, 16), jnp.float32)],
        mesh=_sc_mesh(),
        compiler_params=pltpu.CompilerParams(needs_layout_passes=False, use_tc_tiling_on_sc=False),
        scratch_types=[pltpu.VMEM((NVERT, 4), jnp.float32),
                       pltpu.VMEM((FPW * 3,), jnp.int32),
                       pltpu.VMEM((FPW, 4), jnp.float32),
                       pltpu.VMEM((4, FPW), jnp.float32),
                       pltpu.VMEM((FPW, 16), jnp.float32)],
    )


def _make_knn(bodies, nface, qblk, interpret=False):
    nchunk = nface // 128
    qs = 32

    def _knn_body(qmat_ref, centT_ref, nbr_ref, val_ref):
        qb = pl.program_id(1)
        q = qmat_ref[0]
        c = centT_ref[0]
        val_ref[...] = lax.dot_general(
            q, c, (((1,), (0,)), ((), ())),
            preferred_element_type=jnp.float32)
        eye = (lax.broadcasted_iota(jnp.int32, (qblk, qblk), 0)
               == lax.broadcasted_iota(jnp.int32, (qblk, qblk), 1))
        blk = val_ref[:, pl.ds(qb * qblk, qblk)]
        val_ref[:, pl.ds(qb * qblk, qblk)] = jnp.where(eye, BIG, blk)

        laneiota = lax.broadcasted_iota(jnp.int32, (qs, 128), 1)
        initm = jnp.full((qs, 128), BIG, jnp.float32)
        inita = jnp.zeros((qs, 128), jnp.int32)

        def per_tile(rg, carry):
            def scan_step(t, st):
                m1, m2, m3, a1, a2, a3 = st
                v = val_ref[pl.ds(rg * qs, qs), pl.ds(t * 128, 128)]
                gt = jnp.full((qs, 128), t, jnp.int32)
                c1 = v < m1
                t1 = jnp.maximum(m1, v)
                p1 = jnp.where(c1, a1, gt)
                m1 = jnp.minimum(m1, v)
                a1 = jnp.where(c1, gt, a1)
                c2 = t1 < m2
                t2 = jnp.maximum(m2, t1)
                p2 = jnp.where(c2, a2, p1)
                m2 = jnp.minimum(m2, t1)
                a2 = jnp.where(c2, p1, a2)
                c3 = t2 < m3
                m3 = jnp.minimum(m3, t2)
                a3 = jnp.where(c3, p2, a3)
                return m1, m2, m3, a1, a2, a3

            m1, m2, m3, a1, a2, a3 = lax.fori_loop(
                0, nchunk, scan_step,
                (initm, initm, initm, inita, inita, inita))

            for t in range(KNBR):
                colfull = a1 * 128 + laneiota
                rowmin = jnp.min(m1, axis=1, keepdims=True)
                cand = jnp.where(m1 <= rowmin, colfull, 1 << 30)
                colv = jnp.min(cand, axis=1)
                nbr_ref[0, pl.ds(rg * qs, qs), t] = colv
                sel = colfull == colv[:, None]
                m1 = jnp.where(sel, m2, m1)
                a1 = jnp.where(sel, a2, a1)
                m2 = jnp.where(sel, m3, m2)
                a2 = jnp.where(sel, a3, a2)
                m3 = jnp.where(sel, BIG, m3)
            return carry

        lax.fori_loop(0, qblk // qs, per_tile, 0)

    return pl.pallas_call(
        _knn_body,
        grid=(bodies, nface // qblk),
        in_specs=[pl.BlockSpec((1, qblk, 4), lambda b, qb: (b, qb, 0)),
                  pl.BlockSpec((1, 4, nface), lambda b, qb: (b, 0, 0))],
        out_specs=pl.BlockSpec((1, qblk, KNBR), lambda b, qb: (b, qb, 0)),
        out_shape=jax.ShapeDtypeStruct((bodies, nface, KNBR), jnp.int32),
        scratch_shapes=[pltpu.VMEM((qblk, nface), jnp.float32)],
        interpret=interpret,
    )


@functools.cache
def _knn():
    return _make_knn(BODIES, NFACE, QBLK)


def _field_body(tris_hbm, nbr_hbm, out_hbm, nbuf, obuf, ibuf, gbuf, accbuf,
                sem):
    w = lax.axis_index("s") * NC + lax.axis_index("c")
    f0 = w * FPW
    lane = lax.iota(jnp.int32, LANES)
    magic = jnp.full((LANES,), 0x5F3759DF, jnp.int32)

    def per_body(b, acc):
        pltpu.sync_copy(nbr_hbm.at[b, pl.ds(f0, FPW)], nbuf)
        pltpu.sync_copy(tris_hbm.at[pl.ds(b * NFACE + f0, FPW)], obuf)

        def build(gi, carry):
            for k in range(KNBR):
                g = plsc.load_gather(
                    nbuf, [gi * LANES + lane, jnp.full((LANES,), k, jnp.int32)])
                plsc.store_scatter(
                    ibuf, [jnp.full((LANES,), gi, jnp.int32), k * LANES + lane],
                    g + b * NFACE)
            return carry

        lax.fori_loop(0, NGRP, build, 0)

        descs = [pltpu.async_copy(tris_hbm.at[ibuf.at[gi]], gbuf.at[gi], sem)
                 for gi in range(NGRP)]
        for d in descs:
            d.wait()

        def cgroup(gi, acc2):
            gfull = jnp.full((LANES,), gi, jnp.int32)
            oc = [plsc.load_gather(
                      obuf, [gi * LANES + lane, jnp.full((LANES,), 12 + c, jnp.int32)])
                  for c in range(3)]
            for k in range(KNBR):
                slot = k * LANES + lane
                for j in range(3):
                    s = jnp.full((LANES,), 1e-12, jnp.float32)
                    for c in range(3):
                        gval = plsc.load_gather(
                            gbuf, [gfull, slot,
                                   jnp.full((LANES,), 3 * j + c, jnp.int32)])
                        d0 = gval - oc[c]
                        s = s + d0 * d0
                    sb = plsc.bitcast(s, jnp.int32)
                    y = plsc.bitcast(magic - lax.shift_right_logical(sb, 1),
                                     jnp.float32)
                    y = y * (1.5 - 0.5 * s * y * y)
                    y = y * (1.5 - 0.5 * s * y * y)
                    y = y * (1.5 - 0.5 * s * y * y)
                    dist = s * y
                    fld = jnp.maximum(SIGMA - dist, 0.0)
                    acc2 = acc2 + fld * fld
            return acc2

        return lax.fori_loop(0, NGRP, cgroup, acc)

    acc = lax.fori_loop(0, BODIES, per_body, jnp.zeros((LANES,), jnp.float32))
    accbuf[...] = acc
    pltpu.sync_copy(accbuf, out_hbm.at[w])


@functools.cache
def _field():
    return pl.kernel(
        _field_body,
        out_type=pltpu.HBM((NW, LANES), jnp.float32),
        mesh=_sc_mesh(),
        compiler_params=pltpu.CompilerParams(needs_layout_passes=False, use_tc_tiling_on_sc=False),
        scratch_types=[pltpu.VMEM((FPW, KNBR), jnp.int32),
                       pltpu.VMEM((FPW, 16), jnp.float32),
                       pltpu.VMEM((NGRP, 128), jnp.int32),
                       pltpu.VMEM((NGRP, 128, 16), jnp.float32),
                       pltpu.VMEM((LANES,), jnp.float32),
                       pltpu.SemaphoreType.DMA],
    )


def kernel(vertices, faces):
    vertsP = jnp.pad(vertices, ((0, 0), (0, 0), (0, 1)))
    qmat, centT, tris = _prep()(vertsP, faces)
    nbr = _knn()(qmat, centT)
    parts = _field()(tris, nbr)
    return COLL_W * jnp.sum(parts)


# speedup vs baseline: 1.0225x; 1.0225x over previous
import functools

import jax
import jax.numpy as jnp
from jax import lax
from jax.experimental import pallas as pl
from jax.experimental.pallas import tpu as pltpu
from jax.experimental.pallas import tpu_sc as plsc

BODIES = 16
NVERT = 4096
NFACE = 8192
KNBR = 8
SIGMA = 0.5
COLL_W = 1.0

NC, NS, LANES = 2, 16, 16
NW = NC * NS
FPW = NFACE // NW
NGRP = FPW // LANES
QBLK = 256
BIG = 1e30


def _sc_mesh():
    return plsc.VectorSubcoreMesh(
        core_axis_name="c", subcore_axis_name="s",
        num_cores=NC, num_subcores=NS)


def _prep_body(verts_hbm, faces_hbm, qmat_hbm, centT_hbm, tris_hbm,
               vbuf, fbuf, qbuf, ctbuf, tbuf):
    w = lax.axis_index("s") * NC + lax.axis_index("c")
    f0 = w * FPW
    lane = lax.iota(jnp.int32, LANES)
    one = jnp.full((LANES,), 1.0, jnp.float32)

    pltpu.sync_copy(faces_hbm.at[pl.ds(f0 * 3, FPW * 3)], fbuf)

    def per_body(b, carry):
        pltpu.sync_copy(verts_hbm.at[b], vbuf)

        def per_group(gi, carry2):
            fl = gi * LANES + lane
            vcoord = []
            for j in range(3):
                ij = plsc.load_gather(fbuf, [fl * 3 + j])
                vcoord.append([
                    plsc.load_gather(vbuf, [ij, jnp.full((LANES,), c, jnp.int32)])
                    for c in range(3)])
            cent = [(vcoord[0][c] + vcoord[1][c] + vcoord[2][c]) * (1.0 / 3.0)
                    for c in range(3)]
            nc2 = 0.5 * (cent[0] * cent[0] + cent[1] * cent[1]
                         + cent[2] * cent[2])
            for col, v in enumerate([-cent[0], -cent[1], -cent[2], one]):
                plsc.store_scatter(
                    qbuf, [fl, jnp.full((LANES,), col, jnp.int32)], v)
            for r, v in enumerate([cent[0], cent[1], cent[2], nc2]):
                plsc.store_scatter(
                    ctbuf, [jnp.full((LANES,), r, jnp.int32), fl], v)
            for j in range(3):
                for c in range(3):
                    plsc.store_scatter(
                        tbuf, [fl, jnp.full((LANES,), 3 * j + c, jnp.int32)],
                        vcoord[j][c])
            for c in range(3):
                plsc.store_scatter(
                    tbuf, [fl, jnp.full((LANES,), 12 + c, jnp.int32)], cent[c])
            return carry2

        lax.fori_loop(0, NGRP, per_group, 0)
        pltpu.sync_copy(qbuf, qmat_hbm.at[b, pl.ds(f0, FPW)])
        pltpu.sync_copy(ctbuf, centT_hbm.at[b, :, pl.ds(f0, FPW)])
        pltpu.sync_copy(tbuf, tris_hbm.at[pl.ds(b * NFACE + f0, FPW)])
        return carry

    lax.fori_loop(0, BODIES, per_body, 0)


@functools.cache
def _prep():
    return pl.kernel(
        _prep_body,
        out_type=[pltpu.HBM((BODIES, NFACE, 4), jnp.float32),
                  pltpu.HBM((BODIES, 4, NFACE), jnp.float32),
                  pltpu.HBM((BODIES * NFACE, 16), jnp.float32)],
        mesh=_sc_mesh(),
        compiler_params=pltpu.CompilerParams(needs_layout_passes=False, use_tc_tiling_on_sc=False),
        scratch_types=[pltpu.VMEM((NVERT, 4), jnp.float32),
                       pltpu.VMEM((FPW * 3,), jnp.int32),
                       pltpu.VMEM((FPW, 4), jnp.float32),
                       pltpu.VMEM((4, FPW), jnp.float32),
                       pltpu.VMEM((FPW, 16), jnp.float32)],
    )


def _make_knn(bodies, nface, qblk, interpret=False):
    nchunk = nface // 128
    qs = 32

    def _knn_body(qmat_ref, centT_ref, nbr_ref, val_ref):
        qb = pl.program_id(1)
        q = qmat_ref[0]
        c = centT_ref[0]
        val_ref[...] = lax.dot_general(
            q, c, (((1,), (0,)), ((), ())),
            preferred_element_type=jnp.float32)
        eye = (lax.broadcasted_iota(jnp.int32, (qblk, qblk), 0)
               == lax.broadcasted_iota(jnp.int32, (qblk, qblk), 1))
        blk = val_ref[:, pl.ds(qb * qblk, qblk)]
        val_ref[:, pl.ds(qb * qblk, qblk)] = jnp.where(eye, BIG, blk)

        laneiota = lax.broadcasted_iota(jnp.int32, (qs, 128), 1)
        initm = jnp.full((qs, 128), BIG, jnp.float32)
        inita = jnp.zeros((qs, 128), jnp.int32)

        for rg in range(qblk // qs):
            def scan_step(t, st, rg=rg):
                m1, m2, m3, a1, a2, a3 = st
                v = val_ref[pl.ds(rg * qs, qs), pl.ds(t * 128, 128)]
                gt = jnp.full((qs, 128), t, jnp.int32)
                c1 = v < m1
                t1 = jnp.maximum(m1, v)
                p1 = jnp.where(c1, a1, gt)
                m1 = jnp.minimum(m1, v)
                a1 = jnp.where(c1, gt, a1)
                c2 = t1 < m2
                t2 = jnp.maximum(m2, t1)
                p2 = jnp.where(c2, a2, p1)
                m2 = jnp.minimum(m2, t1)
                a2 = jnp.where(c2, p1, a2)
                c3 = t2 < m3
                m3 = jnp.minimum(m3, t2)
                a3 = jnp.where(c3, p2, a3)
                return m1, m2, m3, a1, a2, a3

            m1, m2, m3, a1, a2, a3 = lax.fori_loop(
                0, nchunk, scan_step,
                (initm, initm, initm, inita, inita, inita))

            for t in range(KNBR):
                colfull = a1 * 128 + laneiota
                rowmin = jnp.min(m1, axis=1, keepdims=True)
                cand = jnp.where(m1 <= rowmin, colfull, 1 << 30)
                colv = jnp.min(cand, axis=1)
                nbr_ref[0, rg * qs:(rg + 1) * qs, t] = colv
                sel = colfull == colv[:, None]
                m1 = jnp.where(sel, m2, m1)
                a1 = jnp.where(sel, a2, a1)
                m2 = jnp.where(sel, m3, m2)
                a2 = jnp.where(sel, a3, a2)
                m3 = jnp.where(sel, BIG, m3)

    return pl.pallas_call(
        _knn_body,
        grid=(bodies, nface // qblk),
        in_specs=[pl.BlockSpec((1, qblk, 4), lambda b, qb: (b, qb, 0)),
                  pl.BlockSpec((1, 4, nface), lambda b, qb: (b, 0, 0))],
        out_specs=pl.BlockSpec((1, qblk, KNBR), lambda b, qb: (b, qb, 0)),
        out_shape=jax.ShapeDtypeStruct((bodies, nface, KNBR), jnp.int32),
        scratch_shapes=[pltpu.VMEM((qblk, nface), jnp.float32)],
        interpret=interpret,
    )


@functools.cache
def _knn():
    return _make_knn(BODIES, NFACE, QBLK)


def _field_body(tris_hbm, nbr_hbm, out_hbm, nbuf, obuf, ibuf, gbuf, accbuf,
                sem):
    w = lax.axis_index("s") * NC + lax.axis_index("c")
    f0 = w * FPW
    lane = lax.iota(jnp.int32, LANES)
    magic = jnp.full((LANES,), 0x5F3759DF, jnp.int32)

    def per_body(b, acc):
        pltpu.sync_copy(nbr_hbm.at[b, pl.ds(f0, FPW)], nbuf)
        pltpu.sync_copy(tris_hbm.at[pl.ds(b * NFACE + f0, FPW)], obuf)

        def build(gi, carry):
            for k in range(KNBR):
                g = plsc.load_gather(
                    nbuf, [gi * LANES + lane, jnp.full((LANES,), k, jnp.int32)])
                plsc.store_scatter(
                    ibuf, [jnp.full((LANES,), gi, jnp.int32), k * LANES + lane],
                    g + b * NFACE)
            return carry

        lax.fori_loop(0, NGRP, build, 0)

        descs = [pltpu.async_copy(tris_hbm.at[ibuf.at[gi]], gbuf.at[gi], sem)
                 for gi in range(NGRP)]
        for d in descs:
            d.wait()

        def cgroup(gi, acc2):
            gfull = jnp.full((LANES,), gi, jnp.int32)
            oc = [plsc.load_gather(
                      obuf, [gi * LANES + lane, jnp.full((LANES,), 12 + c, jnp.int32)])
                  for c in range(3)]
            for k in range(KNBR):
                slot = k * LANES + lane
                for j in range(3):
                    s = jnp.full((LANES,), 1e-12, jnp.float32)
                    for c in range(3):
                        gval = plsc.load_gather(
                            gbuf, [gfull, slot,
                                   jnp.full((LANES,), 3 * j + c, jnp.int32)])
                        d0 = gval - oc[c]
                        s = s + d0 * d0
                    sb = plsc.bitcast(s, jnp.int32)
                    y = plsc.bitcast(magic - lax.shift_right_logical(sb, 1),
                                     jnp.float32)
                    y = y * (1.5 - 0.5 * s * y * y)
                    y = y * (1.5 - 0.5 * s * y * y)
                    y = y * (1.5 - 0.5 * s * y * y)
                    dist = s * y
                    fld = jnp.maximum(SIGMA - dist, 0.0)
                    acc2 = acc2 + fld * fld
            return acc2

        return lax.fori_loop(0, NGRP, cgroup, acc)

    acc = lax.fori_loop(0, BODIES, per_body, jnp.zeros((LANES,), jnp.float32))
    accbuf[...] = acc
    pltpu.sync_copy(accbuf, out_hbm.at[w])


@functools.cache
def _field():
    return pl.kernel(
        _field_body,
        out_type=pltpu.HBM((NW, LANES), jnp.float32),
        mesh=_sc_mesh(),
        compiler_params=pltpu.CompilerParams(needs_layout_passes=False, use_tc_tiling_on_sc=False),
        scratch_types=[pltpu.VMEM((FPW, KNBR), jnp.int32),
                       pltpu.VMEM((FPW, 16), jnp.float32),
                       pltpu.VMEM((NGRP, 128), jnp.int32),
                       pltpu.VMEM((NGRP, 128, 16), jnp.float32),
                       pltpu.VMEM((LANES,), jnp.float32),
                       pltpu.SemaphoreType.DMA],
    )


def kernel(vertices, faces):
    vertsP = jnp.pad(vertices, ((0, 0), (0, 0), (0, 1)))
    qmat, centT, tris = _prep()(vertsP, faces)
    nbr = _knn()(qmat, centT)
    parts = _field()(tris, nbr)
    return COLL_W * jnp.sum(parts)


# speedup vs baseline: 1.5387x; 1.5048x over previous
import functools

import jax
import jax.numpy as jnp
from jax import lax
from jax.experimental import pallas as pl
from jax.experimental.pallas import tpu as pltpu
from jax.experimental.pallas import tpu_sc as plsc

BODIES = 16
NVERT = 4096
NFACE = 8192
KNBR = 8
SIGMA = 0.5
COLL_W = 1.0

NC, NS, LANES = 2, 16, 16
NW = NC * NS
FPW = NFACE // NW
NGRP = FPW // LANES
QBLK = 256
BIG = 1e30


def _sc_mesh():
    return plsc.VectorSubcoreMesh(
        core_axis_name="c", subcore_axis_name="s",
        num_cores=NC, num_subcores=NS)


def _prep_body(verts_hbm, faces_hbm, qmat_hbm, centT_hbm, tris_hbm,
               vbuf, fbuf, qbuf, ctbuf, tbuf):
    w = lax.axis_index("s") * NC + lax.axis_index("c")
    f0 = w * FPW
    lane = lax.iota(jnp.int32, LANES)
    one = jnp.full((LANES,), 1.0, jnp.float32)

    pltpu.sync_copy(faces_hbm.at[pl.ds(f0 * 3, FPW * 3)], fbuf)

    def per_body(b, carry):
        pltpu.sync_copy(verts_hbm.at[b], vbuf)

        def per_group(gi, carry2):
            fl = gi * LANES + lane
            vcoord = []
            for j in range(3):
                ij = plsc.load_gather(fbuf, [fl * 3 + j])
                vcoord.append([
                    plsc.load_gather(vbuf, [ij, jnp.full((LANES,), c, jnp.int32)])
                    for c in range(3)])
            cent = [(vcoord[0][c] + vcoord[1][c] + vcoord[2][c]) * (1.0 / 3.0)
                    for c in range(3)]
            nc2 = 0.5 * (cent[0] * cent[0] + cent[1] * cent[1]
                         + cent[2] * cent[2])
            for col, v in enumerate([-cent[0], -cent[1], -cent[2], one]):
                plsc.store_scatter(
                    qbuf, [fl, jnp.full((LANES,), col, jnp.int32)], v)
            for r, v in enumerate([cent[0], cent[1], cent[2], nc2]):
                plsc.store_scatter(
                    ctbuf, [jnp.full((LANES,), r, jnp.int32), fl], v)
            for j in range(3):
                for c in range(3):
                    plsc.store_scatter(
                        tbuf, [fl, jnp.full((LANES,), 3 * j + c, jnp.int32)],
                        vcoord[j][c])
            for c in range(3):
                plsc.store_scatter(
                    tbuf, [fl, jnp.full((LANES,), 12 + c, jnp.int32)], cent[c])
            return carry2

        lax.fori_loop(0, NGRP, per_group, 0)
        pltpu.sync_copy(qbuf, qmat_hbm.at[b, pl.ds(f0, FPW)])
        pltpu.sync_copy(ctbuf, centT_hbm.at[b, :, pl.ds(f0, FPW)])
        pltpu.sync_copy(tbuf, tris_hbm.at[pl.ds(b * NFACE + f0, FPW)])
        return carry

    lax.fori_loop(0, BODIES, per_body, 0)


@functools.cache
def _prep():
    return pl.kernel(
        _prep_body,
        out_type=[pltpu.HBM((BODIES, NFACE, 4), jnp.float32),
                  pltpu.HBM((BODIES, 4, NFACE), jnp.float32),
                  pltpu.HBM((BODIES * NFACE, 16), jnp.float32)],
        mesh=_sc_mesh(),
        compiler_params=pltpu.CompilerParams(needs_layout_passes=False, use_tc_tiling_on_sc=False),
        scratch_types=[pltpu.VMEM((NVERT, 4), jnp.float32),
                       pltpu.VMEM((FPW * 3,), jnp.int32),
                       pltpu.VMEM((FPW, 4), jnp.float32),
                       pltpu.VMEM((4, FPW), jnp.float32),
                       pltpu.VMEM((FPW, 16), jnp.float32)],
    )


def _make_knn(bodies, nface, qblk, interpret=False):
    nchunk = nface // 128
    qs = 32

    def _knn_body(qmat_ref, centT_ref, nbr_ref, val_ref):
        qb = pl.program_id(1)
        q = qmat_ref[0]
        c = centT_ref[0]
        val_ref[...] = lax.dot_general(
            q, c, (((1,), (0,)), ((), ())),
            preferred_element_type=jnp.float32)
        eye = (lax.broadcasted_iota(jnp.int32, (qblk, qblk), 0)
               == lax.broadcasted_iota(jnp.int32, (qblk, qblk), 1))
        blk = val_ref[:, pl.ds(qb * qblk, qblk)]
        val_ref[:, pl.ds(qb * qblk, qblk)] = jnp.where(eye, BIG, blk)

        laneiota = lax.broadcasted_iota(jnp.int32, (qblk, 128), 1)
        initm = jnp.full((qblk, 128), BIG, jnp.float32)
        inita = jnp.zeros((qblk, 128), jnp.int32)

        def scan_step(t, st):
            m1, m2, m3, a1, a2, a3 = st
            v = val_ref[:, pl.ds(t * 128, 128)]
            gt = jnp.full((qblk, 128), t, jnp.int32)
            c1 = v < m1
            t1 = jnp.maximum(m1, v)
            p1 = jnp.where(c1, a1, gt)
            m1 = jnp.minimum(m1, v)
            a1 = jnp.where(c1, gt, a1)
            c2 = t1 < m2
            t2 = jnp.maximum(m2, t1)
            p2 = jnp.where(c2, a2, p1)
            m2 = jnp.minimum(m2, t1)
            a2 = jnp.where(c2, p1, a2)
            c3 = t2 < m3
            m3 = jnp.minimum(m3, t2)
            a3 = jnp.where(c3, p2, a3)
            return m1, m2, m3, a1, a2, a3

        m1, m2, m3, a1, a2, a3 = lax.fori_loop(
            0, nchunk, scan_step,
            (initm, initm, initm, inita, inita, inita))

        for t in range(KNBR):
            colfull = a1 * 128 + laneiota
            rowmin = jnp.min(m1, axis=1, keepdims=True)
            cand = jnp.where(m1 <= rowmin, colfull, 1 << 30)
            colv = jnp.min(cand, axis=1)
            nbr_ref[0, :, t] = colv
            sel = colfull == colv[:, None]
            m1 = jnp.where(sel, m2, m1)
            a1 = jnp.where(sel, a2, a1)
            m2 = jnp.where(sel, m3, m2)
            a2 = jnp.where(sel, a3, a2)
            m3 = jnp.where(sel, BIG, m3)

    return pl.pallas_call(
        _knn_body,
        grid=(bodies, nface // qblk),
        in_specs=[pl.BlockSpec((1, qblk, 4), lambda b, qb: (b, qb, 0)),
                  pl.BlockSpec((1, 4, nface), lambda b, qb: (b, 0, 0))],
        out_specs=pl.BlockSpec((1, qblk, KNBR), lambda b, qb: (b, qb, 0)),
        out_shape=jax.ShapeDtypeStruct((bodies, nface, KNBR), jnp.int32),
        scratch_shapes=[pltpu.VMEM((qblk, nface), jnp.float32)],
        interpret=interpret,
    )


@functools.cache
def _knn():
    return _make_knn(BODIES, NFACE, QBLK)


def _field_body(tris_hbm, nbr_hbm, out_hbm, nbuf, obuf, ibuf, gbuf, accbuf,
                sem):
    w = lax.axis_index("s") * NC + lax.axis_index("c")
    f0 = w * FPW
    lane = lax.iota(jnp.int32, LANES)
    magic = jnp.full((LANES,), 0x5F3759DF, jnp.int32)

    def per_body(b, acc):
        pltpu.sync_copy(nbr_hbm.at[b, pl.ds(f0, FPW)], nbuf)
        pltpu.sync_copy(tris_hbm.at[pl.ds(b * NFACE + f0, FPW)], obuf)

        def build(gi, carry):
            for k in range(KNBR):
                g = plsc.load_gather(
                    nbuf, [gi * LANES + lane, jnp.full((LANES,), k, jnp.int32)])
                plsc.store_scatter(
                    ibuf, [jnp.full((LANES,), gi, jnp.int32), k * LANES + lane],
                    g + b * NFACE)
            return carry

        lax.fori_loop(0, NGRP, build, 0)

        descs = [pltpu.async_copy(tris_hbm.at[ibuf.at[gi]], gbuf.at[gi], sem)
                 for gi in range(NGRP)]
        for d in descs:
            d.wait()

        def cgroup(gi, acc2):
            gfull = jnp.full((LANES,), gi, jnp.int32)
            oc = [plsc.load_gather(
                      obuf, [gi * LANES + lane, jnp.full((LANES,), 12 + c, jnp.int32)])
                  for c in range(3)]
            for k in range(KNBR):
                slot = k * LANES + lane
                for j in range(3):
                    s = jnp.full((LANES,), 1e-12, jnp.float32)
                    for c in range(3):
                        gval = plsc.load_gather(
                            gbuf, [gfull, slot,
                                   jnp.full((LANES,), 3 * j + c, jnp.int32)])
                        d0 = gval - oc[c]
                        s = s + d0 * d0
                    sb = plsc.bitcast(s, jnp.int32)
                    y = plsc.bitcast(magic - lax.shift_right_logical(sb, 1),
                                     jnp.float32)
                    y = y * (1.5 - 0.5 * s * y * y)
                    y = y * (1.5 - 0.5 * s * y * y)
                    y = y * (1.5 - 0.5 * s * y * y)
                    dist = s * y
                    fld = jnp.maximum(SIGMA - dist, 0.0)
                    acc2 = acc2 + fld * fld
            return acc2

        return lax.fori_loop(0, NGRP, cgroup, acc)

    acc = lax.fori_loop(0, BODIES, per_body, jnp.zeros((LANES,), jnp.float32))
    accbuf[...] = acc
    pltpu.sync_copy(accbuf, out_hbm.at[w])


@functools.cache
def _field():
    return pl.kernel(
        _field_body,
        out_type=pltpu.HBM((NW, LANES), jnp.float32),
        mesh=_sc_mesh(),
        compiler_params=pltpu.CompilerParams(needs_layout_passes=False, use_tc_tiling_on_sc=False),
        scratch_types=[pltpu.VMEM((FPW, KNBR), jnp.int32),
                       pltpu.VMEM((FPW, 16), jnp.float32),
                       pltpu.VMEM((NGRP, 128), jnp.int32),
                       pltpu.VMEM((NGRP, 128, 16), jnp.float32),
                       pltpu.VMEM((LANES,), jnp.float32),
                       pltpu.SemaphoreType.DMA],
    )


def kernel(vertices, faces):
    vertsP = jnp.pad(vertices, ((0, 0), (0, 0), (0, 1)))
    qmat, centT, tris = _prep()(vertsP, faces)
    nbr = _knn()(qmat, centT)
    parts = _field()(tris, nbr)
    return COLL_W * jnp.sum(parts)


# speedup vs baseline: 1.9216x; 1.2488x over previous
import functools

import jax
import jax.numpy as jnp
from jax import lax
from jax.experimental import pallas as pl
from jax.experimental.pallas import tpu as pltpu
from jax.experimental.pallas import tpu_sc as plsc

BODIES = 16
NVERT = 4096
NFACE = 8192
KNBR = 8
SIGMA = 0.5
COLL_W = 1.0

NC, NS, LANES = 2, 16, 16
NW = NC * NS
FPW = NFACE // NW
NGRP = FPW // LANES
QBLK = 256
BIG = 1e30


def _sc_mesh():
    return plsc.VectorSubcoreMesh(
        core_axis_name="c", subcore_axis_name="s",
        num_cores=NC, num_subcores=NS)


def _prep_body(verts_hbm, faces_hbm, qmat_hbm, centT_hbm, tris_hbm,
               vbuf, fbuf, qbuf, ctbuf, tbuf):
    w = lax.axis_index("s") * NC + lax.axis_index("c")
    f0 = w * FPW
    lane = lax.iota(jnp.int32, LANES)
    one = jnp.full((LANES,), 1.0, jnp.float32)

    pltpu.sync_copy(faces_hbm.at[pl.ds(f0 * 3, FPW * 3)], fbuf)

    def per_body(b, carry):
        pltpu.sync_copy(verts_hbm.at[b], vbuf)

        def per_group(gi, carry2):
            fl = gi * LANES + lane
            vcoord = []
            for j in range(3):
                ij = plsc.load_gather(fbuf, [fl * 3 + j])
                vcoord.append([
                    plsc.load_gather(vbuf, [ij, jnp.full((LANES,), c, jnp.int32)])
                    for c in range(3)])
            cent = [(vcoord[0][c] + vcoord[1][c] + vcoord[2][c]) * (1.0 / 3.0)
                    for c in range(3)]
            nc2 = 0.5 * (cent[0] * cent[0] + cent[1] * cent[1]
                         + cent[2] * cent[2])
            for col, v in enumerate([-cent[0], -cent[1], -cent[2], one]):
                plsc.store_scatter(
                    qbuf, [fl, jnp.full((LANES,), col, jnp.int32)], v)
            for r, v in enumerate([cent[0], cent[1], cent[2], nc2]):
                plsc.store_scatter(
                    ctbuf, [jnp.full((LANES,), r, jnp.int32), fl], v)
            for j in range(3):
                for c in range(3):
                    plsc.store_scatter(
                        tbuf, [fl, jnp.full((LANES,), 3 * j + c, jnp.int32)],
                        vcoord[j][c])
            for c in range(3):
                plsc.store_scatter(
                    tbuf, [fl, jnp.full((LANES,), 12 + c, jnp.int32)], cent[c])
            return carry2

        lax.fori_loop(0, NGRP, per_group, 0)
        pltpu.sync_copy(qbuf, qmat_hbm.at[b, pl.ds(f0, FPW)])
        pltpu.sync_copy(ctbuf, centT_hbm.at[b, :, pl.ds(f0, FPW)])
        pltpu.sync_copy(tbuf, tris_hbm.at[pl.ds(b * NFACE + f0, FPW)])
        return carry

    lax.fori_loop(0, BODIES, per_body, 0)


@functools.cache
def _prep():
    return pl.kernel(
        _prep_body,
        out_type=[pltpu.HBM((BODIES, NFACE, 4), jnp.float32),
                  pltpu.HBM((BODIES, 4, NFACE), jnp.float32),
                  pltpu.HBM((BODIES * NFACE, 16), jnp.float32)],
        mesh=_sc_mesh(),
        compiler_params=pltpu.CompilerParams(needs_layout_passes=False, use_tc_tiling_on_sc=False),
        scratch_types=[pltpu.VMEM((NVERT, 4), jnp.float32),
                       pltpu.VMEM((FPW * 3,), jnp.int32),
                       pltpu.VMEM((FPW, 4), jnp.float32),
                       pltpu.VMEM((4, FPW), jnp.float32),
                       pltpu.VMEM((FPW, 16), jnp.float32)],
    )


def _make_knn(bodies, nface, qblk, interpret=False):
    nchunk = nface // 128
    qs = 32

    def _knn_body(qmat_ref, centT_ref, nbr_ref, val_ref):
        qb = pl.program_id(1)
        q = qmat_ref[0]
        c = centT_ref[0]
        val_ref[...] = lax.dot_general(
            q, c, (((1,), (0,)), ((), ())),
            preferred_element_type=jnp.float32)
        eye = (lax.broadcasted_iota(jnp.int32, (qblk, qblk), 0)
               == lax.broadcasted_iota(jnp.int32, (qblk, qblk), 1))
        blk = val_ref[:, pl.ds(qb * qblk, qblk)]
        val_ref[:, pl.ds(qb * qblk, qblk)] = jnp.where(eye, BIG, blk)

        laneiota = lax.broadcasted_iota(jnp.int32, (qblk, 128), 1)
        initm = jnp.full((qblk, 128), BIG, jnp.float32)
        inita = jnp.zeros((qblk, 128), jnp.int32)

        unroll = 4

        def insert_one(st, t):
            m1, m2, m3, a1, a2, a3 = st
            v = val_ref[:, pl.ds(t * 128, 128)]
            gt = jnp.full((qblk, 128), t, jnp.int32)
            c1 = v < m1
            t1 = jnp.maximum(m1, v)
            p1 = jnp.where(c1, a1, gt)
            m1 = jnp.minimum(m1, v)
            a1 = jnp.where(c1, gt, a1)
            c2 = t1 < m2
            t2 = jnp.maximum(m2, t1)
            p2 = jnp.where(c2, a2, p1)
            m2 = jnp.minimum(m2, t1)
            a2 = jnp.where(c2, p1, a2)
            c3 = t2 < m3
            m3 = jnp.minimum(m3, t2)
            a3 = jnp.where(c3, p2, a3)
            return m1, m2, m3, a1, a2, a3

        def scan_step(i, st):
            for u in range(unroll):
                st = insert_one(st, i * unroll + u)
            return st

        m1, m2, m3, a1, a2, a3 = lax.fori_loop(
            0, nchunk // unroll, scan_step,
            (initm, initm, initm, inita, inita, inita))

        for t in range(KNBR):
            colfull = a1 * 128 + laneiota
            rowmin = jnp.min(m1, axis=1, keepdims=True)
            cand = jnp.where(m1 <= rowmin, colfull, 1 << 30)
            colv = jnp.min(cand, axis=1)
            nbr_ref[0, :, t] = colv
            sel = colfull == colv[:, None]
            m1 = jnp.where(sel, m2, m1)
            a1 = jnp.where(sel, a2, a1)
            m2 = jnp.where(sel, m3, m2)
            a2 = jnp.where(sel, a3, a2)
            m3 = jnp.where(sel, BIG, m3)

    return pl.pallas_call(
        _knn_body,
        grid=(bodies, nface // qblk),
        in_specs=[pl.BlockSpec((1, qblk, 4), lambda b, qb: (b, qb, 0)),
                  pl.BlockSpec((1, 4, nface), lambda b, qb: (b, 0, 0))],
        out_specs=pl.BlockSpec((1, qblk, KNBR), lambda b, qb: (b, qb, 0)),
        out_shape=jax.ShapeDtypeStruct((bodies, nface, KNBR), jnp.int32),
        scratch_shapes=[pltpu.VMEM((qblk, nface), jnp.float32)],
        interpret=interpret,
    )


@functools.cache
def _knn():
    return _make_knn(BODIES, NFACE, QBLK)


def _field_body(tris_hbm, nbr_hbm, out_hbm, nbuf, obuf, ibuf, gbuf, accbuf,
                sem):
    w = lax.axis_index("s") * NC + lax.axis_index("c")
    f0 = w * FPW
    lane = lax.iota(jnp.int32, LANES)
    magic = jnp.full((LANES,), 0x5F3759DF, jnp.int32)

    def per_body(b, acc):
        pltpu.sync_copy(nbr_hbm.at[b, pl.ds(f0, FPW)], nbuf)
        pltpu.sync_copy(tris_hbm.at[pl.ds(b * NFACE + f0, FPW)], obuf)

        def build(gi, carry):
            for k in range(KNBR):
                g = plsc.load_gather(
                    nbuf, [gi * LANES + lane, jnp.full((LANES,), k, jnp.int32)])
                plsc.store_scatter(
                    ibuf, [jnp.full((LANES,), gi, jnp.int32), k * LANES + lane],
                    g + b * NFACE)
            return carry

        lax.fori_loop(0, NGRP, build, 0)

        descs = [pltpu.async_copy(tris_hbm.at[ibuf.at[gi]], gbuf.at[gi], sem)
                 for gi in range(NGRP)]
        for d in descs:
            d.wait()

        def cgroup(gi, acc2):
            gfull = jnp.full((LANES,), gi, jnp.int32)
            oc = [plsc.load_gather(
                      obuf, [gi * LANES + lane, jnp.full((LANES,), 12 + c, jnp.int32)])
                  for c in range(3)]
            for k in range(KNBR):
                slot = k * LANES + lane
                for j in range(3):
                    s = jnp.full((LANES,), 1e-12, jnp.float32)
                    for c in range(3):
                        gval = plsc.load_gather(
                            gbuf, [gfull, slot,
                                   jnp.full((LANES,), 3 * j + c, jnp.int32)])
                        d0 = gval - oc[c]
                        s = s + d0 * d0
                    sb = plsc.bitcast(s, jnp.int32)
                    y = plsc.bitcast(magic - lax.shift_right_logical(sb, 1),
                                     jnp.float32)
                    y = y * (1.5 - 0.5 * s * y * y)
                    y = y * (1.5 - 0.5 * s * y * y)
                    y = y * (1.5 - 0.5 * s * y * y)
                    dist = s * y
                    fld = jnp.maximum(SIGMA - dist, 0.0)
                    acc2 = acc2 + fld * fld
            return acc2

        return lax.fori_loop(0, NGRP, cgroup, acc)

    acc = lax.fori_loop(0, BODIES, per_body, jnp.zeros((LANES,), jnp.float32))
    accbuf[...] = acc
    pltpu.sync_copy(accbuf, out_hbm.at[w])


@functools.cache
def _field():
    return pl.kernel(
        _field_body,
        out_type=pltpu.HBM((NW, LANES), jnp.float32),
        mesh=_sc_mesh(),
        compiler_params=pltpu.CompilerParams(needs_layout_passes=False, use_tc_tiling_on_sc=False),
        scratch_types=[pltpu.VMEM((FPW, KNBR), jnp.int32),
                       pltpu.VMEM((FPW, 16), jnp.float32),
                       pltpu.VMEM((NGRP, 128), jnp.int32),
                       pltpu.VMEM((NGRP, 128, 16), jnp.float32),
                       pltpu.VMEM((LANES,), jnp.float32),
                       pltpu.SemaphoreType.DMA],
    )


def kernel(vertices, faces):
    vertsP = jnp.pad(vertices, ((0, 0), (0, 0), (0, 1)))
    qmat, centT, tris = _prep()(vertsP, faces)
    nbr = _knn()(qmat, centT)
    parts = _field()(tris, nbr)
    return COLL_W * jnp.sum(parts)


# speedup vs baseline: 2.0250x; 1.0538x over previous
import functools

import jax
import jax.numpy as jnp
from jax import lax
from jax.experimental import pallas as pl
from jax.experimental.pallas import tpu as pltpu
from jax.experimental.pallas import tpu_sc as plsc

BODIES = 16
NVERT = 4096
NFACE = 8192
KNBR = 8
SIGMA = 0.5
COLL_W = 1.0

NC, NS, LANES = 2, 16, 16
NW = NC * NS
FPW = NFACE // NW
NGRP = FPW // LANES
QBLK = 256
BIG = 1e30


def _sc_mesh():
    return plsc.VectorSubcoreMesh(
        core_axis_name="c", subcore_axis_name="s",
        num_cores=NC, num_subcores=NS)


def _prep_body(verts_hbm, faces_hbm, qmat_hbm, centT_hbm, tris_hbm,
               vbuf, fbuf, qbuf, ctbuf, tbuf):
    w = lax.axis_index("s") * NC + lax.axis_index("c")
    f0 = w * FPW
    lane = lax.iota(jnp.int32, LANES)
    one = jnp.full((LANES,), 1.0, jnp.float32)

    pltpu.sync_copy(faces_hbm.at[pl.ds(f0 * 3, FPW * 3)], fbuf)

    def per_body(b, carry):
        pltpu.sync_copy(verts_hbm.at[b], vbuf)

        def per_group(gi, carry2):
            fl = gi * LANES + lane
            vcoord = []
            for j in range(3):
                ij = plsc.load_gather(fbuf, [fl * 3 + j])
                vcoord.append([
                    plsc.load_gather(vbuf, [ij, jnp.full((LANES,), c, jnp.int32)])
                    for c in range(3)])
            cent = [(vcoord[0][c] + vcoord[1][c] + vcoord[2][c]) * (1.0 / 3.0)
                    for c in range(3)]
            nc2 = 0.5 * (cent[0] * cent[0] + cent[1] * cent[1]
                         + cent[2] * cent[2])
            for col, v in enumerate([-cent[0], -cent[1], -cent[2], one]):
                plsc.store_scatter(
                    qbuf, [fl, jnp.full((LANES,), col, jnp.int32)], v)
            for r, v in enumerate([cent[0], cent[1], cent[2], nc2]):
                plsc.store_scatter(
                    ctbuf, [jnp.full((LANES,), r, jnp.int32), fl], v)
            for j in range(3):
                for c in range(3):
                    plsc.store_scatter(
                        tbuf, [fl, jnp.full((LANES,), 3 * j + c, jnp.int32)],
                        vcoord[j][c])
            for c in range(3):
                plsc.store_scatter(
                    tbuf, [fl, jnp.full((LANES,), 12 + c, jnp.int32)], cent[c])
            return carry2

        lax.fori_loop(0, NGRP, per_group, 0)
        pltpu.sync_copy(qbuf, qmat_hbm.at[b, pl.ds(f0, FPW)])
        pltpu.sync_copy(ctbuf, centT_hbm.at[b, :, pl.ds(f0, FPW)])
        pltpu.sync_copy(tbuf, tris_hbm.at[pl.ds(b * NFACE + f0, FPW)])
        return carry

    lax.fori_loop(0, BODIES, per_body, 0)


@functools.cache
def _prep():
    return pl.kernel(
        _prep_body,
        out_type=[pltpu.HBM((BODIES, NFACE, 4), jnp.float32),
                  pltpu.HBM((BODIES, 4, NFACE), jnp.float32),
                  pltpu.HBM((BODIES * NFACE, 16), jnp.float32)],
        mesh=_sc_mesh(),
        compiler_params=pltpu.CompilerParams(needs_layout_passes=False, use_tc_tiling_on_sc=False),
        scratch_types=[pltpu.VMEM((NVERT, 4), jnp.float32),
                       pltpu.VMEM((FPW * 3,), jnp.int32),
                       pltpu.VMEM((FPW, 4), jnp.float32),
                       pltpu.VMEM((4, FPW), jnp.float32),
                       pltpu.VMEM((FPW, 16), jnp.float32)],
    )


def _make_knn(bodies, nface, qblk, interpret=False):
    nchunk = nface // 128
    qs = 32

    def _knn_body(qmat_ref, centT_ref, nbr_ref, val_ref):
        qb = pl.program_id(1)
        q = qmat_ref[0]
        c = centT_ref[0]
        val_ref[...] = lax.dot_general(
            q, c, (((1,), (0,)), ((), ())),
            preferred_element_type=jnp.float32)
        eye = (lax.broadcasted_iota(jnp.int32, (qblk, qblk), 0)
               == lax.broadcasted_iota(jnp.int32, (qblk, qblk), 1))
        blk = val_ref[:, pl.ds(qb * qblk, qblk)]
        val_ref[:, pl.ds(qb * qblk, qblk)] = jnp.where(eye, BIG, blk)

        laneiota = lax.broadcasted_iota(jnp.int32, (qblk, 128), 1)
        initm = jnp.full((qblk, 128), BIG, jnp.float32)
        inita = jnp.zeros((qblk, 128), jnp.int32)

        unroll = 8

        def insert_one(st, t):
            m1, m2, m3, a1, a2, a3 = st
            v = val_ref[:, pl.ds(t * 128, 128)]
            gt = jnp.full((qblk, 128), t, jnp.int32)
            c1 = v < m1
            t1 = jnp.maximum(m1, v)
            p1 = jnp.where(c1, a1, gt)
            m1 = jnp.minimum(m1, v)
            a1 = jnp.where(c1, gt, a1)
            c2 = t1 < m2
            t2 = jnp.maximum(m2, t1)
            p2 = jnp.where(c2, a2, p1)
            m2 = jnp.minimum(m2, t1)
            a2 = jnp.where(c2, p1, a2)
            c3 = t2 < m3
            m3 = jnp.minimum(m3, t2)
            a3 = jnp.where(c3, p2, a3)
            return m1, m2, m3, a1, a2, a3

        def scan_step(i, st):
            for u in range(unroll):
                st = insert_one(st, i * unroll + u)
            return st

        m1, m2, m3, a1, a2, a3 = lax.fori_loop(
            0, nchunk // unroll, scan_step,
            (initm, initm, initm, inita, inita, inita))

        for t in range(KNBR):
            colfull = a1 * 128 + laneiota
            rowmin = jnp.min(m1, axis=1, keepdims=True)
            cand = jnp.where(m1 <= rowmin, colfull, 1 << 30)
            colv = jnp.min(cand, axis=1)
            nbr_ref[0, :, t] = colv
            sel = colfull == colv[:, None]
            m1 = jnp.where(sel, m2, m1)
            a1 = jnp.where(sel, a2, a1)
            m2 = jnp.where(sel, m3, m2)
            a2 = jnp.where(sel, a3, a2)
            m3 = jnp.where(sel, BIG, m3)

    return pl.pallas_call(
        _knn_body,
        grid=(bodies, nface // qblk),
        in_specs=[pl.BlockSpec((1, qblk, 4), lambda b, qb: (b, qb, 0)),
                  pl.BlockSpec((1, 4, nface), lambda b, qb: (b, 0, 0))],
        out_specs=pl.BlockSpec((1, qblk, KNBR), lambda b, qb: (b, qb, 0)),
        out_shape=jax.ShapeDtypeStruct((bodies, nface, KNBR), jnp.int32),
        scratch_shapes=[pltpu.VMEM((qblk, nface), jnp.float32)],
        interpret=interpret,
    )


@functools.cache
def _knn():
    return _make_knn(BODIES, NFACE, QBLK)


def _field_body(tris_hbm, nbr_hbm, out_hbm, nbuf, obuf, ibuf, gbuf, accbuf,
                sem):
    w = lax.axis_index("s") * NC + lax.axis_index("c")
    f0 = w * FPW
    lane = lax.iota(jnp.int32, LANES)
    magic = jnp.full((LANES,), 0x5F3759DF, jnp.int32)

    def per_body(b, acc):
        pltpu.sync_copy(nbr_hbm.at[b, pl.ds(f0, FPW)], nbuf)
        pltpu.sync_copy(tris_hbm.at[pl.ds(b * NFACE + f0, FPW)], obuf)

        def build(gi, carry):
            for k in range(KNBR):
                g = plsc.load_gather(
                    nbuf, [gi * LANES + lane, jnp.full((LANES,), k, jnp.int32)])
                plsc.store_scatter(
                    ibuf, [jnp.full((LANES,), gi, jnp.int32), k * LANES + lane],
                    g + b * NFACE)
            return carry

        lax.fori_loop(0, NGRP, build, 0)

        descs = [pltpu.async_copy(tris_hbm.at[ibuf.at[gi]], gbuf.at[gi], sem)
                 for gi in range(NGRP)]
        for d in descs:
            d.wait()

        def cgroup(gi, acc2):
            gfull = jnp.full((LANES,), gi, jnp.int32)
            oc = [plsc.load_gather(
                      obuf, [gi * LANES + lane, jnp.full((LANES,), 12 + c, jnp.int32)])
                  for c in range(3)]
            for k in range(KNBR):
                slot = k * LANES + lane
                for j in range(3):
                    s = jnp.full((LANES,), 1e-12, jnp.float32)
                    for c in range(3):
                        gval = plsc.load_gather(
                            gbuf, [gfull, slot,
                                   jnp.full((LANES,), 3 * j + c, jnp.int32)])
                        d0 = gval - oc[c]
                        s = s + d0 * d0
                    sb = plsc.bitcast(s, jnp.int32)
                    y = plsc.bitcast(magic - lax.shift_right_logical(sb, 1),
                                     jnp.float32)
                    y = y * (1.5 - 0.5 * s * y * y)
                    y = y * (1.5 - 0.5 * s * y * y)
                    y = y * (1.5 - 0.5 * s * y * y)
                    dist = s * y
                    fld = jnp.maximum(SIGMA - dist, 0.0)
                    acc2 = acc2 + fld * fld
            return acc2

        return lax.fori_loop(0, NGRP, cgroup, acc)

    acc = lax.fori_loop(0, BODIES, per_body, jnp.zeros((LANES,), jnp.float32))
    accbuf[...] = acc
    pltpu.sync_copy(accbuf, out_hbm.at[w])


@functools.cache
def _field():
    return pl.kernel(
        _field_body,
        out_type=pltpu.HBM((NW, LANES), jnp.float32),
        mesh=_sc_mesh(),
        compiler_params=pltpu.CompilerParams(needs_layout_passes=False, use_tc_tiling_on_sc=False),
        scratch_types=[pltpu.VMEM((FPW, KNBR), jnp.int32),
                       pltpu.VMEM((FPW, 16), jnp.float32),
                       pltpu.VMEM((NGRP, 128), jnp.int32),
                       pltpu.VMEM((NGRP, 128, 16), jnp.float32),
                       pltpu.VMEM((LANES,), jnp.float32),
                       pltpu.SemaphoreType.DMA],
    )


def kernel(vertices, faces):
    vertsP = jnp.pad(vertices, ((0, 0), (0, 0), (0, 1)))
    qmat, centT, tris = _prep()(vertsP, faces)
    nbr = _knn()(qmat, centT)
    parts = _field()(tris, nbr)
    return COLL_W * jnp.sum(parts)


# speedup vs baseline: 2.0634x; 1.0190x over previous
import functools

import jax
import jax.numpy as jnp
from jax import lax
from jax.experimental import pallas as pl
from jax.experimental.pallas import tpu as pltpu
from jax.experimental.pallas import tpu_sc as plsc

BODIES = 16
NVERT = 4096
NFACE = 8192
KNBR = 8
SIGMA = 0.5
COLL_W = 1.0

NC, NS, LANES = 2, 16, 16
NW = NC * NS
FPW = NFACE // NW
NGRP = FPW // LANES
QBLK = 256
BIG = 1e30


def _sc_mesh():
    return plsc.VectorSubcoreMesh(
        core_axis_name="c", subcore_axis_name="s",
        num_cores=NC, num_subcores=NS)


def _prep_body(verts_hbm, faces_hbm, qmat_hbm, centT_hbm, tris_hbm,
               vbuf, fbuf, qbuf, ctbuf, tbuf):
    w = lax.axis_index("s") * NC + lax.axis_index("c")
    f0 = w * FPW
    lane = lax.iota(jnp.int32, LANES)
    one = jnp.full((LANES,), 1.0, jnp.float32)

    pltpu.sync_copy(faces_hbm.at[pl.ds(f0 * 3, FPW * 3)], fbuf)

    def per_body(b, carry):
        pltpu.sync_copy(verts_hbm.at[b], vbuf)

        def per_group(gi, carry2):
            fl = gi * LANES + lane
            vcoord = []
            for j in range(3):
                ij = plsc.load_gather(fbuf, [fl * 3 + j])
                vcoord.append([
                    plsc.load_gather(vbuf, [ij, jnp.full((LANES,), c, jnp.int32)])
                    for c in range(3)])
            cent = [(vcoord[0][c] + vcoord[1][c] + vcoord[2][c]) * (1.0 / 3.0)
                    for c in range(3)]
            nc2 = 0.5 * (cent[0] * cent[0] + cent[1] * cent[1]
                         + cent[2] * cent[2])
            for col, v in enumerate([-cent[0], -cent[1], -cent[2], one]):
                plsc.store_scatter(
                    qbuf, [fl, jnp.full((LANES,), col, jnp.int32)], v)
            for r, v in enumerate([cent[0], cent[1], cent[2], nc2]):
                plsc.store_scatter(
                    ctbuf, [jnp.full((LANES,), r, jnp.int32), fl], v)
            for j in range(3):
                for c in range(3):
                    plsc.store_scatter(
                        tbuf, [fl, jnp.full((LANES,), 3 * j + c, jnp.int32)],
                        vcoord[j][c])
            for c in range(3):
                plsc.store_scatter(
                    tbuf, [fl, jnp.full((LANES,), 12 + c, jnp.int32)], cent[c])
            return carry2

        lax.fori_loop(0, NGRP, per_group, 0)
        pltpu.sync_copy(qbuf, qmat_hbm.at[b, pl.ds(f0, FPW)])
        pltpu.sync_copy(ctbuf, centT_hbm.at[b, :, pl.ds(f0, FPW)])
        pltpu.sync_copy(tbuf, tris_hbm.at[pl.ds(b * NFACE + f0, FPW)])
        return carry

    lax.fori_loop(0, BODIES, per_body, 0)


@functools.cache
def _prep():
    return pl.kernel(
        _prep_body,
        out_type=[pltpu.HBM((BODIES, NFACE, 4), jnp.float32),
                  pltpu.HBM((BODIES, 4, NFACE), jnp.float32),
                  pltpu.HBM((BODIES * NFACE, 16), jnp.float32)],
        mesh=_sc_mesh(),
        compiler_params=pltpu.CompilerParams(needs_layout_passes=False, use_tc_tiling_on_sc=False),
        scratch_types=[pltpu.VMEM((NVERT, 4), jnp.float32),
                       pltpu.VMEM((FPW * 3,), jnp.int32),
                       pltpu.VMEM((FPW, 4), jnp.float32),
                       pltpu.VMEM((4, FPW), jnp.float32),
                       pltpu.VMEM((FPW, 16), jnp.float32)],
    )


def _make_knn(bodies, nface, qblk, interpret=False):
    nchunk = nface // 128
    qs = 32

    def _knn_body(qmat_ref, centT_ref, nbr_ref, val_ref):
        qb = pl.program_id(1)
        q = qmat_ref[0]
        c = centT_ref[0]
        val_ref[...] = lax.dot_general(
            q, c, (((1,), (0,)), ((), ())),
            preferred_element_type=jnp.float32)
        eye = (lax.broadcasted_iota(jnp.int32, (qblk, qblk), 0)
               == lax.broadcasted_iota(jnp.int32, (qblk, qblk), 1))
        blk = val_ref[:, pl.ds(qb * qblk, qblk)]
        val_ref[:, pl.ds(qb * qblk, qblk)] = jnp.where(eye, BIG, blk)

        laneiota = lax.broadcasted_iota(jnp.int32, (qblk, 128), 1)
        initm = jnp.full((qblk, 128), BIG, jnp.float32)
        inita = jnp.zeros((qblk, 128), jnp.int32)

        unroll = 16

        def insert_one(st, t):
            m1, m2, m3, a1, a2, a3 = st
            v = val_ref[:, pl.ds(t * 128, 128)]
            gt = jnp.full((qblk, 128), t, jnp.int32)
            c1 = v < m1
            t1 = jnp.maximum(m1, v)
            p1 = jnp.where(c1, a1, gt)
            m1 = jnp.minimum(m1, v)
            a1 = jnp.where(c1, gt, a1)
            c2 = t1 < m2
            t2 = jnp.maximum(m2, t1)
            p2 = jnp.where(c2, a2, p1)
            m2 = jnp.minimum(m2, t1)
            a2 = jnp.where(c2, p1, a2)
            c3 = t2 < m3
            m3 = jnp.minimum(m3, t2)
            a3 = jnp.where(c3, p2, a3)
            return m1, m2, m3, a1, a2, a3

        def scan_step(i, st):
            for u in range(unroll):
                st = insert_one(st, i * unroll + u)
            return st

        m1, m2, m3, a1, a2, a3 = lax.fori_loop(
            0, nchunk // unroll, scan_step,
            (initm, initm, initm, inita, inita, inita))

        for t in range(KNBR):
            colfull = a1 * 128 + laneiota
            rowmin = jnp.min(m1, axis=1, keepdims=True)
            cand = jnp.where(m1 <= rowmin, colfull, 1 << 30)
            colv = jnp.min(cand, axis=1)
            nbr_ref[0, :, t] = colv
            sel = colfull == colv[:, None]
            m1 = jnp.where(sel, m2, m1)
            a1 = jnp.where(sel, a2, a1)
            m2 = jnp.where(sel, m3, m2)
            a2 = jnp.where(sel, a3, a2)
            m3 = jnp.where(sel, BIG, m3)

    return pl.pallas_call(
        _knn_body,
        grid=(bodies, nface // qblk),
        in_specs=[pl.BlockSpec((1, qblk, 4), lambda b, qb: (b, qb, 0)),
                  pl.BlockSpec((1, 4, nface), lambda b, qb: (b, 0, 0))],
        out_specs=pl.BlockSpec((1, qblk, KNBR), lambda b, qb: (b, qb, 0)),
        out_shape=jax.ShapeDtypeStruct((bodies, nface, KNBR), jnp.int32),
        scratch_shapes=[pltpu.VMEM((qblk, nface), jnp.float32)],
        interpret=interpret,
    )


@functools.cache
def _knn():
    return _make_knn(BODIES, NFACE, QBLK)


def _field_body(tris_hbm, nbr_hbm, out_hbm, nbuf, obuf, ibuf, gbuf, accbuf,
                sem):
    w = lax.axis_index("s") * NC + lax.axis_index("c")
    f0 = w * FPW
    lane = lax.iota(jnp.int32, LANES)
    magic = jnp.full((LANES,), 0x5F3759DF, jnp.int32)

    def per_body(b, acc):
        pltpu.sync_copy(nbr_hbm.at[b, pl.ds(f0, FPW)], nbuf)
        pltpu.sync_copy(tris_hbm.at[pl.ds(b * NFACE + f0, FPW)], obuf)

        def build(gi, carry):
            for k in range(KNBR):
                g = plsc.load_gather(
                    nbuf, [gi * LANES + lane, jnp.full((LANES,), k, jnp.int32)])
                plsc.store_scatter(
                    ibuf, [jnp.full((LANES,), gi, jnp.int32), k * LANES + lane],
                    g + b * NFACE)
            return carry

        lax.fori_loop(0, NGRP, build, 0)

        descs = [pltpu.async_copy(tris_hbm.at[ibuf.at[gi]], gbuf.at[gi], sem)
                 for gi in range(NGRP)]
        for d in descs:
            d.wait()

        def cgroup(gi, acc2):
            gfull = jnp.full((LANES,), gi, jnp.int32)
            oc = [plsc.load_gather(
                      obuf, [gi * LANES + lane, jnp.full((LANES,), 12 + c, jnp.int32)])
                  for c in range(3)]
            for k in range(KNBR):
                slot = k * LANES + lane
                for j in range(3):
                    s = jnp.full((LANES,), 1e-12, jnp.float32)
                    for c in range(3):
                        gval = plsc.load_gather(
                            gbuf, [gfull, slot,
                                   jnp.full((LANES,), 3 * j + c, jnp.int32)])
                        d0 = gval - oc[c]
                        s = s + d0 * d0
                    sb = plsc.bitcast(s, jnp.int32)
                    y = plsc.bitcast(magic - lax.shift_right_logical(sb, 1),
                                     jnp.float32)
                    y = y * (1.5 - 0.5 * s * y * y)
                    y = y * (1.5 - 0.5 * s * y * y)
                    y = y * (1.5 - 0.5 * s * y * y)
                    dist = s * y
                    fld = jnp.maximum(SIGMA - dist, 0.0)
                    acc2 = acc2 + fld * fld
            return acc2

        return lax.fori_loop(0, NGRP, cgroup, acc)

    acc = lax.fori_loop(0, BODIES, per_body, jnp.zeros((LANES,), jnp.float32))
    accbuf[...] = acc
    pltpu.sync_copy(accbuf, out_hbm.at[w])


@functools.cache
def _field():
    return pl.kernel(
        _field_body,
        out_type=pltpu.HBM((NW, LANES), jnp.float32),
        mesh=_sc_mesh(),
        compiler_params=pltpu.CompilerParams(needs_layout_passes=False, use_tc_tiling_on_sc=False),
        scratch_types=[pltpu.VMEM((FPW, KNBR), jnp.int32),
                       pltpu.VMEM((FPW, 16), jnp.float32),
                       pltpu.VMEM((NGRP, 128), jnp.int32),
                       pltpu.VMEM((NGRP, 128, 16), jnp.float32),
                       pltpu.VMEM((LANES,), jnp.float32),
                       pltpu.SemaphoreType.DMA],
    )


def kernel(vertices, faces):
    vertsP = jnp.pad(vertices, ((0, 0), (0, 0), (0, 1)))
    qmat, centT, tris = _prep()(vertsP, faces)
    nbr = _knn()(qmat, centT)
    parts = _field()(tris, nbr)
    return COLL_W * jnp.sum(parts)
